# baseline (speedup 1.0000x reference)
import jax
import jax.numpy as jnp
from jax.experimental import pallas as pl
from jax.experimental.pallas import tpu as pltpu

NSEQ = 8
BEAM = 3
HID = 768
VOCAB = 30522
SEQ = 512
SEP_TOKEN = 102
QMARK_TOKEN = 1029

ROWS = NSEQ * BEAM
VT = 8192
NTILES = (VOCAB + VT - 1) // VT
NEG = -1e30

_DN = (((1,), (1,)), ((), ()))


def _proj_topk_kernel(hid_ref, wt_ref, b_ref, p_ref, idx_ref,
                      m_ref, s_ref, v3_ref, i3_ref):
    j = pl.program_id(0)

    @pl.when(j == 0)
    def _init():
        m_ref[:] = jnp.full((ROWS, 1), NEG, jnp.float32)
        s_ref[:] = jnp.zeros((ROWS, 1), jnp.float32)
        v3_ref[:] = jnp.full((ROWS, BEAM), NEG, jnp.float32)
        i3_ref[:] = jnp.zeros((ROWS, BEAM), jnp.int32)

    x = jax.lax.dot_general(hid_ref[:], wt_ref[:], _DN,
                            preferred_element_type=jnp.float32) + b_ref[:]
    lane = jax.lax.broadcasted_iota(jnp.int32, (ROWS, VT), 1)
    x = jnp.where(lane + j * VT < VOCAB, x, NEG)

    tmax = jnp.max(x, axis=1, keepdims=True)
    m_old = m_ref[:]
    m_new = jnp.maximum(m_old, tmax)
    s_ref[:] = (s_ref[:] * jnp.exp(m_old - m_new)
                + jnp.sum(jnp.exp(x - m_new), axis=1, keepdims=True))
    m_ref[:] = m_new

    work = x
    tv, ti = [], []
    for _ in range(BEAM):
        mx = jnp.max(work, axis=1, keepdims=True)
        sel = jnp.min(jnp.where(work == mx, lane, VT), axis=1, keepdims=True)
        tv.append(mx)
        ti.append(sel + j * VT)
        work = jnp.where(lane == sel, NEG, work)

    vals6 = jnp.concatenate([v3_ref[:]] + tv, axis=1)
    idx6 = jnp.concatenate([i3_ref[:]] + ti, axis=1)
    lane6 = jax.lax.broadcasted_iota(jnp.int32, (ROWS, 2 * BEAM), 1)
    mv, mi = [], []
    for _ in range(BEAM):
        mx = jnp.max(vals6, axis=1, keepdims=True)
        sel = jnp.min(jnp.where(vals6 == mx, lane6, 2 * BEAM), axis=1, keepdims=True)
        gi = jnp.sum(jnp.where(lane6 == sel, idx6, 0), axis=1, keepdims=True)
        mv.append(mx)
        mi.append(gi)
        vals6 = jnp.where(lane6 == sel, NEG, vals6)
    v3_ref[:] = jnp.concatenate(mv, axis=1)
    i3_ref[:] = jnp.concatenate(mi, axis=1)

    @pl.when(j == NTILES - 1)
    def _fin():
        p_ref[:] = jnp.exp(v3_ref[:] - m_ref[:]) / s_ref[:]
        idx_ref[:] = i3_ref[:]


def _beam_merge_kernel(p9_ref, t9_ref, bp_ref, b0_ref, b1_ref, b2_ref,
                       nb0_ref, nb1_ref, nb2_ref, np_ref, gen_ref):
    p9 = p9_ref[:]
    t9 = t9_ref[:]
    bp = bp_ref[:]
    ep = jnp.concatenate(
        [bp[:, i:i + 1] * p9[:, 3 * i:3 * i + 3] for i in range(BEAM)], axis=1)
    lane9 = jax.lax.broadcasted_iota(jnp.int32, (NSEQ, BEAM * BEAM), 1)
    beams = (b0_ref[:], b1_ref[:], b2_ref[:])
    nb_refs = (nb0_ref, nb1_ref, nb2_ref)
    work = ep
    for i in range(BEAM):
        mx = jnp.max(work, axis=1, keepdims=True)
        sel = jnp.min(jnp.where(work == mx, lane9, BEAM * BEAM), axis=1, keepdims=True)
        tok = jnp.sum(jnp.where(lane9 == sel, t9, 0), axis=1, keepdims=True)
        parent = sel // BEAM
        work = jnp.where(lane9 == sel, -1.0, work)
        pb = jnp.where(parent == 0, beams[0], jnp.where(parent == 1, beams[1], beams[2]))
        nb_refs[i][:, 0:SEQ] = pb
        nb_refs[i][:, pl.ds(SEQ, 1)] = tok
        sep = (jnp.sum((pb == SEP_TOKEN).astype(jnp.int32), axis=1, keepdims=True)
               + (tok == SEP_TOKEN).astype(jnp.int32))
        qm = (jnp.sum((pb == QMARK_TOKEN).astype(jnp.int32), axis=1, keepdims=True)
              + (tok == QMARK_TOKEN).astype(jnp.int32))
        gen = jnp.logical_and(sep <= 3, qm == 0).astype(jnp.int32)
        np_ref[:, pl.ds(i, 1)] = mx
        gen_ref[:, pl.ds(i, 1)] = gen


def kernel(hidden, W_sqg, b_sqg, beams, beam_probs):
    p3, idx3 = pl.pallas_call(
        _proj_topk_kernel,
        grid=(NTILES,),
        in_specs=[
            pl.BlockSpec((ROWS, HID), lambda j: (0, 0)),
            pl.BlockSpec((VT, HID), lambda j: (j, 0)),
            pl.BlockSpec((1, VT), lambda j: (0, j)),
        ],
        out_specs=[
            pl.BlockSpec((ROWS, BEAM), lambda j: (0, 0)),
            pl.BlockSpec((ROWS, BEAM), lambda j: (0, 0)),
        ],
        out_shape=[
            jax.ShapeDtypeStruct((ROWS, BEAM), jnp.float32),
            jax.ShapeDtypeStruct((ROWS, BEAM), jnp.int32),
        ],
        scratch_shapes=[
            pltpu.VMEM((ROWS, 1), jnp.float32),
            pltpu.VMEM((ROWS, 1), jnp.float32),
            pltpu.VMEM((ROWS, BEAM), jnp.float32),
            pltpu.VMEM((ROWS, BEAM), jnp.int32),
        ],
        compiler_params=pltpu.CompilerParams(dimension_semantics=("arbitrary",)),
    )(hidden, W_sqg.T, b_sqg)

    beams3 = beams.reshape(NSEQ, BEAM, SEQ)
    nb0, nb1, nb2, np3, gen3 = pl.pallas_call(
        _beam_merge_kernel,
        out_shape=[
            jax.ShapeDtypeStruct((NSEQ, SEQ + 1), jnp.int32),
            jax.ShapeDtypeStruct((NSEQ, SEQ + 1), jnp.int32),
            jax.ShapeDtypeStruct((NSEQ, SEQ + 1), jnp.int32),
            jax.ShapeDtypeStruct((NSEQ, BEAM), jnp.float32),
            jax.ShapeDtypeStruct((NSEQ, BEAM), jnp.int32),
        ],
    )(p3.reshape(NSEQ, BEAM * BEAM), idx3.reshape(NSEQ, BEAM * BEAM),
      beam_probs.reshape(NSEQ, BEAM),
      beams3[:, 0, :], beams3[:, 1, :], beams3[:, 2, :])

    new_beams = jnp.stack([nb0, nb1, nb2], axis=1).reshape(ROWS, SEQ + 1)
    return new_beams, np3.reshape(ROWS), gen3.reshape(ROWS) != 0


# speedup vs baseline: 1.0043x; 1.0043x over previous
import jax
import jax.numpy as jnp
from jax.experimental import pallas as pl
from jax.experimental.pallas import tpu as pltpu

NSEQ = 8
BEAM = 3
HID = 768
VOCAB = 30522
SEQ = 512
SEP_TOKEN = 102
QMARK_TOKEN = 1029

ROWS = NSEQ * BEAM
VT = 7680
NTILES = (VOCAB + VT - 1) // VT
NEG = -1e30

_DN = (((1,), (1,)), ((), ()))


def _proj_topk_kernel(hid_ref, wt_ref, b_ref, p_ref, idx_ref,
                      m_ref, s_ref, v3_ref, i3_ref):
    j = pl.program_id(0)

    @pl.when(j == 0)
    def _init():
        m_ref[:] = jnp.full((ROWS, 1), NEG, jnp.float32)
        s_ref[:] = jnp.zeros((ROWS, 1), jnp.float32)
        v3_ref[:] = jnp.full((ROWS, BEAM), NEG, jnp.float32)
        i3_ref[:] = jnp.zeros((ROWS, BEAM), jnp.int32)

    x = jax.lax.dot_general(hid_ref[:], wt_ref[:], _DN,
                            preferred_element_type=jnp.float32) + b_ref[:]
    lane = jax.lax.broadcasted_iota(jnp.int32, (ROWS, VT), 1)
    x = jnp.where(lane + j * VT < VOCAB, x, NEG)

    tmax = jnp.max(x, axis=1, keepdims=True)
    m_old = m_ref[:]
    m_new = jnp.maximum(m_old, tmax)
    s_ref[:] = (s_ref[:] * jnp.exp(m_old - m_new)
                + jnp.sum(jnp.exp(x - m_new), axis=1, keepdims=True))
    m_ref[:] = m_new

    work = x
    tv, ti = [], []
    for _ in range(BEAM):
        mx = jnp.max(work, axis=1, keepdims=True)
        sel = jnp.min(jnp.where(work == mx, lane, VT), axis=1, keepdims=True)
        tv.append(mx)
        ti.append(sel + j * VT)
        work = jnp.where(lane == sel, NEG, work)

    vals6 = jnp.concatenate([v3_ref[:]] + tv, axis=1)
    idx6 = jnp.concatenate([i3_ref[:]] + ti, axis=1)
    lane6 = jax.lax.broadcasted_iota(jnp.int32, (ROWS, 2 * BEAM), 1)
    mv, mi = [], []
    for _ in range(BEAM):
        mx = jnp.max(vals6, axis=1, keepdims=True)
        sel = jnp.min(jnp.where(vals6 == mx, lane6, 2 * BEAM), axis=1, keepdims=True)
        gi = jnp.sum(jnp.where(lane6 == sel, idx6, 0), axis=1, keepdims=True)
        mv.append(mx)
        mi.append(gi)
        vals6 = jnp.where(lane6 == sel, NEG, vals6)
    v3_ref[:] = jnp.concatenate(mv, axis=1)
    i3_ref[:] = jnp.concatenate(mi, axis=1)

    @pl.when(j == NTILES - 1)
    def _fin():
        p_ref[:] = jnp.exp(v3_ref[:] - m_ref[:]) / s_ref[:]
        idx_ref[:] = i3_ref[:]


def _beam_merge_kernel(p9_ref, t9_ref, bp_ref, b0_ref, b1_ref, b2_ref,
                       nb0_ref, nb1_ref, nb2_ref, np_ref, gen_ref):
    p9 = p9_ref[:]
    t9 = t9_ref[:]
    bp = bp_ref[:]
    ep = jnp.concatenate(
        [bp[:, i:i + 1] * p9[:, 3 * i:3 * i + 3] for i in range(BEAM)], axis=1)
    lane9 = jax.lax.broadcasted_iota(jnp.int32, (NSEQ, BEAM * BEAM), 1)
    beams = (b0_ref[:], b1_ref[:], b2_ref[:])
    nb_refs = (nb0_ref, nb1_ref, nb2_ref)
    work = ep
    for i in range(BEAM):
        mx = jnp.max(work, axis=1, keepdims=True)
        sel = jnp.min(jnp.where(work == mx, lane9, BEAM * BEAM), axis=1, keepdims=True)
        tok = jnp.sum(jnp.where(lane9 == sel, t9, 0), axis=1, keepdims=True)
        parent = sel // BEAM
        work = jnp.where(lane9 == sel, -1.0, work)
        pb = jnp.where(parent == 0, beams[0], jnp.where(parent == 1, beams[1], beams[2]))
        nb_refs[i][:, 0:SEQ] = pb
        nb_refs[i][:, pl.ds(SEQ, 1)] = tok
        sep = (jnp.sum((pb == SEP_TOKEN).astype(jnp.int32), axis=1, keepdims=True)
               + (tok == SEP_TOKEN).astype(jnp.int32))
        qm = (jnp.sum((pb == QMARK_TOKEN).astype(jnp.int32), axis=1, keepdims=True)
              + (tok == QMARK_TOKEN).astype(jnp.int32))
        gen = jnp.logical_and(sep <= 3, qm == 0).astype(jnp.int32)
        np_ref[:, pl.ds(i, 1)] = mx
        gen_ref[:, pl.ds(i, 1)] = gen


def kernel(hidden, W_sqg, b_sqg, beams, beam_probs):
    p3, idx3 = pl.pallas_call(
        _proj_topk_kernel,
        grid=(NTILES,),
        in_specs=[
            pl.BlockSpec((ROWS, HID), lambda j: (0, 0)),
            pl.BlockSpec((VT, HID), lambda j: (j, 0)),
            pl.BlockSpec((1, VT), lambda j: (0, j)),
        ],
        out_specs=[
            pl.BlockSpec((ROWS, BEAM), lambda j: (0, 0)),
            pl.BlockSpec((ROWS, BEAM), lambda j: (0, 0)),
        ],
        out_shape=[
            jax.ShapeDtypeStruct((ROWS, BEAM), jnp.float32),
            jax.ShapeDtypeStruct((ROWS, BEAM), jnp.int32),
        ],
        scratch_shapes=[
            pltpu.VMEM((ROWS, 1), jnp.float32),
            pltpu.VMEM((ROWS, 1), jnp.float32),
            pltpu.VMEM((ROWS, BEAM), jnp.float32),
            pltpu.VMEM((ROWS, BEAM), jnp.int32),
        ],
        compiler_params=pltpu.CompilerParams(dimension_semantics=("arbitrary",)),
    )(hidden, W_sqg.T, b_sqg)

    beams3 = beams.reshape(NSEQ, BEAM, SEQ)
    nb0, nb1, nb2, np3, gen3 = pl.pallas_call(
        _beam_merge_kernel,
        out_shape=[
            jax.ShapeDtypeStruct((NSEQ, SEQ + 1), jnp.int32),
            jax.ShapeDtypeStruct((NSEQ, SEQ + 1), jnp.int32),
            jax.ShapeDtypeStruct((NSEQ, SEQ + 1), jnp.int32),
            jax.ShapeDtypeStruct((NSEQ, BEAM), jnp.float32),
            jax.ShapeDtypeStruct((NSEQ, BEAM), jnp.int32),
        ],
    )(p3.reshape(NSEQ, BEAM * BEAM), idx3.reshape(NSEQ, BEAM * BEAM),
      beam_probs.reshape(NSEQ, BEAM),
      beams3[:, 0, :], beams3[:, 1, :], beams3[:, 2, :])

    new_beams = jnp.stack([nb0, nb1, nb2], axis=1).reshape(ROWS, SEQ + 1)
    return new_beams, np3.reshape(ROWS), gen3.reshape(ROWS) != 0


# speedup vs baseline: 1.0465x; 1.0420x over previous
import jax
import jax.numpy as jnp
from jax.experimental import pallas as pl
from jax.experimental.pallas import tpu as pltpu

NSEQ = 8
BEAM = 3
HID = 768
VOCAB = 30522
SEQ = 512
SEP_TOKEN = 102
QMARK_TOKEN = 1029

ROWS = NSEQ * BEAM
VT = 5120
NTILES = (VOCAB + VT - 1) // VT
NEG = -1e30

_DN = (((1,), (1,)), ((), ()))


def _proj_topk_kernel(hid_ref, wt_ref, b_ref, p_ref, idx_ref,
                      m_ref, s_ref, v3_ref, i3_ref):
    j = pl.program_id(0)

    @pl.when(j == 0)
    def _init():
        m_ref[:] = jnp.full((ROWS, 1), NEG, jnp.float32)
        s_ref[:] = jnp.zeros((ROWS, 1), jnp.float32)
        v3_ref[:] = jnp.full((ROWS, BEAM), NEG, jnp.float32)
        i3_ref[:] = jnp.zeros((ROWS, BEAM), jnp.int32)

    x = jax.lax.dot_general(hid_ref[:], wt_ref[:], _DN,
                            preferred_element_type=jnp.float32) + b_ref[:]
    lane = jax.lax.broadcasted_iota(jnp.int32, (ROWS, VT), 1)
    x = jnp.where(lane + j * VT < VOCAB, x, NEG)

    tmax = jnp.max(x, axis=1, keepdims=True)
    m_old = m_ref[:]
    m_new = jnp.maximum(m_old, tmax)
    s_ref[:] = (s_ref[:] * jnp.exp(m_old - m_new)
                + jnp.sum(jnp.exp(x - m_new), axis=1, keepdims=True))
    m_ref[:] = m_new

    work = x
    tv, ti = [], []
    for _ in range(BEAM):
        mx = jnp.max(work, axis=1, keepdims=True)
        sel = jnp.min(jnp.where(work == mx, lane, VT), axis=1, keepdims=True)
        tv.append(mx)
        ti.append(sel + j * VT)
        work = jnp.where(lane == sel, NEG, work)

    vals6 = jnp.concatenate([v3_ref[:]] + tv, axis=1)
    idx6 = jnp.concatenate([i3_ref[:]] + ti, axis=1)
    lane6 = jax.lax.broadcasted_iota(jnp.int32, (ROWS, 2 * BEAM), 1)
    mv, mi = [], []
    for _ in range(BEAM):
        mx = jnp.max(vals6, axis=1, keepdims=True)
        sel = jnp.min(jnp.where(vals6 == mx, lane6, 2 * BEAM), axis=1, keepdims=True)
        gi = jnp.sum(jnp.where(lane6 == sel, idx6, 0), axis=1, keepdims=True)
        mv.append(mx)
        mi.append(gi)
        vals6 = jnp.where(lane6 == sel, NEG, vals6)
    v3_ref[:] = jnp.concatenate(mv, axis=1)
    i3_ref[:] = jnp.concatenate(mi, axis=1)

    @pl.when(j == NTILES - 1)
    def _fin():
        p_ref[:] = jnp.exp(v3_ref[:] - m_ref[:]) / s_ref[:]
        idx_ref[:] = i3_ref[:]


def _beam_merge_kernel(p9_ref, t9_ref, bp_ref, b0_ref, b1_ref, b2_ref,
                       nb0_ref, nb1_ref, nb2_ref, np_ref, gen_ref):
    p9 = p9_ref[:]
    t9 = t9_ref[:]
    bp = bp_ref[:]
    ep = jnp.concatenate(
        [bp[:, i:i + 1] * p9[:, 3 * i:3 * i + 3] for i in range(BEAM)], axis=1)
    lane9 = jax.lax.broadcasted_iota(jnp.int32, (NSEQ, BEAM * BEAM), 1)
    beams = (b0_ref[:], b1_ref[:], b2_ref[:])
    nb_refs = (nb0_ref, nb1_ref, nb2_ref)
    work = ep
    for i in range(BEAM):
        mx = jnp.max(work, axis=1, keepdims=True)
        sel = jnp.min(jnp.where(work == mx, lane9, BEAM * BEAM), axis=1, keepdims=True)
        tok = jnp.sum(jnp.where(lane9 == sel, t9, 0), axis=1, keepdims=True)
        parent = sel // BEAM
        work = jnp.where(lane9 == sel, -1.0, work)
        pb = jnp.where(parent == 0, beams[0], jnp.where(parent == 1, beams[1], beams[2]))
        nb_refs[i][:, 0:SEQ] = pb
        nb_refs[i][:, pl.ds(SEQ, 1)] = tok
        sep = (jnp.sum((pb == SEP_TOKEN).astype(jnp.int32), axis=1, keepdims=True)
               + (tok == SEP_TOKEN).astype(jnp.int32))
        qm = (jnp.sum((pb == QMARK_TOKEN).astype(jnp.int32), axis=1, keepdims=True)
              + (tok == QMARK_TOKEN).astype(jnp.int32))
        gen = jnp.logical_and(sep <= 3, qm == 0).astype(jnp.int32)
        np_ref[:, pl.ds(i, 1)] = mx
        gen_ref[:, pl.ds(i, 1)] = gen


def kernel(hidden, W_sqg, b_sqg, beams, beam_probs):
    p3, idx3 = pl.pallas_call(
        _proj_topk_kernel,
        grid=(NTILES,),
        in_specs=[
            pl.BlockSpec((ROWS, HID), lambda j: (0, 0)),
            pl.BlockSpec((VT, HID), lambda j: (j, 0)),
            pl.BlockSpec((1, VT), lambda j: (0, j)),
        ],
        out_specs=[
            pl.BlockSpec((ROWS, BEAM), lambda j: (0, 0)),
            pl.BlockSpec((ROWS, BEAM), lambda j: (0, 0)),
        ],
        out_shape=[
            jax.ShapeDtypeStruct((ROWS, BEAM), jnp.float32),
            jax.ShapeDtypeStruct((ROWS, BEAM), jnp.int32),
        ],
        scratch_shapes=[
            pltpu.VMEM((ROWS, 1), jnp.float32),
            pltpu.VMEM((ROWS, 1), jnp.float32),
            pltpu.VMEM((ROWS, BEAM), jnp.float32),
            pltpu.VMEM((ROWS, BEAM), jnp.int32),
        ],
        compiler_params=pltpu.CompilerParams(dimension_semantics=("arbitrary",)),
    )(hidden, W_sqg.T, b_sqg)

    beams3 = beams.reshape(NSEQ, BEAM, SEQ)
    nb0, nb1, nb2, np3, gen3 = pl.pallas_call(
        _beam_merge_kernel,
        out_shape=[
            jax.ShapeDtypeStruct((NSEQ, SEQ + 1), jnp.int32),
            jax.ShapeDtypeStruct((NSEQ, SEQ + 1), jnp.int32),
            jax.ShapeDtypeStruct((NSEQ, SEQ + 1), jnp.int32),
            jax.ShapeDtypeStruct((NSEQ, BEAM), jnp.float32),
            jax.ShapeDtypeStruct((NSEQ, BEAM), jnp.int32),
        ],
    )(p3.reshape(NSEQ, BEAM * BEAM), idx3.reshape(NSEQ, BEAM * BEAM),
      beam_probs.reshape(NSEQ, BEAM),
      beams3[:, 0, :], beams3[:, 1, :], beams3[:, 2, :])

    new_beams = jnp.stack([nb0, nb1, nb2], axis=1).reshape(ROWS, SEQ + 1)
    return new_beams, np3.reshape(ROWS), gen3.reshape(ROWS) != 0
